# Initial kernel scaffold; baseline (speedup 1.0000x reference)
import jax
import jax.numpy as jnp
from jax.experimental import pallas as pl


def kernel(position_ids, table):
    raise NotImplementedError("write your pallas kernel here")


# speedup vs baseline: 1.7224x; 1.7224x over previous
import functools
import jax
import jax.numpy as jnp
from jax import lax
from jax.experimental import pallas as pl
from jax.experimental.pallas import tpu as pltpu
from jax.experimental.pallas import tpu_sc as plsc

DIM = 2048
B_TOTAL = 16384
NUM_WORKERS = 32
B_PER_W = B_TOTAL // NUM_WORKERS
CHUNK = 16
NCHUNK = B_PER_W // CHUNK
NBUF = 2

_mesh = plsc.VectorSubcoreMesh(core_axis_name="c", subcore_axis_name="s")


@functools.partial(
    pl.kernel,
    mesh=_mesh,
    out_type=jax.ShapeDtypeStruct((B_TOTAL, DIM), jnp.float32),
    scratch_types=[
        pltpu.VMEM((B_PER_W,), jnp.int32),
        pltpu.VMEM((NBUF, CHUNK, DIM), jnp.float32),
        pltpu.SemaphoreType.DMA,
        pltpu.SemaphoreType.DMA,
    ],
)
def _gather_kernel(table_hbm, idx_hbm, out_hbm, idx_v, rows_v, gsem, ssem):
    wid = lax.axis_index("s") * 2 + lax.axis_index("c")
    base = wid * B_PER_W
    pltpu.sync_copy(idx_hbm.at[pl.ds(base, B_PER_W)], idx_v)

    pltpu.async_copy(
        table_hbm.at[idx_v.at[pl.ds(0, CHUNK)]], rows_v.at[0], gsem
    )

    def outer(c0, _):
        for b in range(NBUF):
            c = c0 * NBUF + b
            pltpu.make_async_copy(
                table_hbm.at[idx_v.at[pl.ds(0, CHUNK)]], rows_v.at[b], gsem
            ).wait()
            nb = (b + 1) % NBUF

            @pl.when(c + 1 < NCHUNK)
            def _():
                @pl.when(c + 1 >= NBUF)
                def _():
                    pltpu.make_async_copy(
                        rows_v.at[nb],
                        out_hbm.at[pl.ds(base + (c + 1 - NBUF) * CHUNK, CHUNK)],
                        ssem,
                    ).wait()

                pltpu.async_copy(
                    table_hbm.at[idx_v.at[pl.ds((c + 1) * CHUNK, CHUNK)]],
                    rows_v.at[nb],
                    gsem,
                )

            pltpu.async_copy(
                rows_v.at[b], out_hbm.at[pl.ds(base + c * CHUNK, CHUNK)], ssem
            )
        return 0

    lax.fori_loop(0, NCHUNK // NBUF, outer, 0)

    for b in range(NBUF):
        c = NCHUNK - NBUF + b
        pltpu.make_async_copy(
            rows_v.at[b % NBUF],
            out_hbm.at[pl.ds(base + c * CHUNK, CHUNK)],
            ssem,
        ).wait()


def kernel(position_ids, table):
    idx = position_ids.astype(jnp.int32).reshape(-1)
    out = _gather_kernel(table, idx)
    return out.reshape(position_ids.shape + (DIM,))


# speedup vs baseline: 1.8052x; 1.0481x over previous
import functools
import jax
import jax.numpy as jnp
from jax import lax
from jax.experimental import pallas as pl
from jax.experimental.pallas import tpu as pltpu
from jax.experimental.pallas import tpu_sc as plsc

DIM = 2048
B_TOTAL = 16384
NUM_WORKERS = 32
B_PER_W = B_TOTAL // NUM_WORKERS
CHUNK = 8
NCHUNK = B_PER_W // CHUNK
NBUF = 4

_mesh = plsc.VectorSubcoreMesh(core_axis_name="c", subcore_axis_name="s")


@functools.partial(
    pl.kernel,
    mesh=_mesh,
    out_type=jax.ShapeDtypeStruct((B_TOTAL, DIM), jnp.float32),
    scratch_types=[
        pltpu.VMEM((B_PER_W,), jnp.int32),
        pltpu.VMEM((NBUF, CHUNK, DIM), jnp.float32),
        pltpu.SemaphoreType.DMA,
        pltpu.SemaphoreType.DMA,
    ],
)
def _gather_kernel(table_hbm, idx_hbm, out_hbm, idx_v, rows_v, gsem, ssem):
    wid = lax.axis_index("s") * 2 + lax.axis_index("c")
    base = wid * B_PER_W
    pltpu.sync_copy(idx_hbm.at[pl.ds(base, B_PER_W)], idx_v)

    for c in range(NBUF - 1):
        pltpu.async_copy(
            table_hbm.at[idx_v.at[pl.ds(c * CHUNK, CHUNK)]],
            rows_v.at[c],
            gsem,
        )

    def outer(i, _):
        for b in range(NBUF):
            c = i * NBUF + b
            pltpu.make_async_copy(
                table_hbm.at[idx_v.at[pl.ds(0, CHUNK)]], rows_v.at[b], gsem
            ).wait()
            g = c + NBUF - 1
            gb = (b - 1) % NBUF

            @pl.when(jnp.logical_and(g < NCHUNK, c >= 1))
            def _():
                pltpu.make_async_copy(
                    rows_v.at[gb],
                    out_hbm.at[pl.ds(base, CHUNK)],
                    ssem,
                ).wait()
                pltpu.async_copy(
                    table_hbm.at[idx_v.at[pl.ds(g * CHUNK, CHUNK)]],
                    rows_v.at[gb],
                    gsem,
                )

            @pl.when(jnp.logical_and(g < NCHUNK, c < 1))
            def _():
                pltpu.async_copy(
                    table_hbm.at[idx_v.at[pl.ds(g * CHUNK, CHUNK)]],
                    rows_v.at[gb],
                    gsem,
                )

            pltpu.async_copy(
                rows_v.at[b], out_hbm.at[pl.ds(base + c * CHUNK, CHUNK)], ssem
            )
        return 0

    lax.fori_loop(0, NCHUNK // NBUF, outer, 0)

    for _ in range(NBUF):
        pltpu.make_async_copy(
            rows_v.at[0],
            out_hbm.at[pl.ds(base, CHUNK)],
            ssem,
        ).wait()


def kernel(position_ids, table):
    idx = position_ids.astype(jnp.int32).reshape(-1)
    out = _gather_kernel(table, idx)
    return out.reshape(position_ids.shape + (DIM,))
